# Initial kernel scaffold; baseline (speedup 1.0000x reference)
import jax
import jax.numpy as jnp
from jax.experimental import pallas as pl


def kernel(x, pos, params, edge_index, batch):
    raise NotImplementedError("write your pallas kernel here")


# speedup vs baseline: 1.6638x; 1.6638x over previous
import functools

import numpy as np
import jax
import jax.numpy as jnp
from jax import lax
from jax.experimental import pallas as pl
from jax.experimental.pallas import tpu as pltpu
from jax.experimental.pallas import tpu_sc as plsc

_N = 10000
_E = 320000
_DIN = 128
_HID = 120
_HP = 128
_SCAL = 32
_RBF = 8
_NSEG = 16
_OUT = 512
_SCALES = (3.0, 6.0, 10.0)
_REPS = (3,) * 16 + (5,) * 8

_NC = 2
_NS = 16
_NW = _NC * _NS
_EW = _E // _NW
_C = 80
_NCH = _EW // _C
_NP = 10240
_RPT = _NP // _NS


def _gate_matrix():
    R = np.zeros((_HP, _HP), np.float32)
    col = _SCAL
    for k, r in enumerate(_REPS):
        for _ in range(r):
            R[k, col] = 1.0
            col += 1
    return jnp.asarray(R)


def _dist2_sc(posx, posy, posz, src, dst):
    mesh = plsc.VectorSubcoreMesh(core_axis_name="c", subcore_axis_name="s")

    @functools.partial(
        pl.kernel,
        out_type=jax.ShapeDtypeStruct((_E,), jnp.float32),
        mesh=mesh,
        scratch_types=[
            pltpu.VMEM((_N,), jnp.float32),
            pltpu.VMEM((_N,), jnp.float32),
            pltpu.VMEM((_N,), jnp.float32),
            pltpu.VMEM((_EW,), jnp.int32),
            pltpu.VMEM((_EW,), jnp.int32),
            pltpu.VMEM((_EW,), jnp.float32),
        ],
        compiler_params=pltpu.CompilerParams(needs_layout_passes=False),
    )
    def k(px_hbm, py_hbm, pz_hbm, src_hbm, dst_hbm, out_hbm, px, py, pz, sv, dv, ov):
        c = lax.axis_index("c")
        s = lax.axis_index("s")
        wid = s * _NC + c
        base = wid * _EW
        pltpu.sync_copy(px_hbm, px)
        pltpu.sync_copy(py_hbm, py)
        pltpu.sync_copy(pz_hbm, pz)
        pltpu.sync_copy(src_hbm.at[pl.ds(base, _EW)], sv)
        pltpu.sync_copy(dst_hbm.at[pl.ds(base, _EW)], dv)

        def body(i, carry):
            off = i * 16
            s16 = sv[pl.ds(off, 16)]
            d16 = dv[pl.ds(off, 16)]
            dx = plsc.load_gather(px, [d16]) - plsc.load_gather(px, [s16])
            dy = plsc.load_gather(py, [d16]) - plsc.load_gather(py, [s16])
            dz = plsc.load_gather(pz, [d16]) - plsc.load_gather(pz, [s16])
            ov[pl.ds(off, 16)] = dx * dx + dy * dy + dz * dz
            return carry

        lax.fori_loop(0, _EW // 16, body, 0)
        pltpu.sync_copy(ov, out_hbm.at[pl.ds(base, _EW)])

    return k(posx, posy, posz, src, dst)


def _edge_sc(h, wenv, src, dst):
    mesh = plsc.VectorSubcoreMesh(core_axis_name="c", subcore_axis_name="s")

    @functools.partial(
        pl.kernel,
        out_type=jax.ShapeDtypeStruct((_NC, _NP, _HP), jnp.float32),
        mesh=mesh,
        scratch_types=[
            pltpu.VMEM_SHARED((_NP, _HP), jnp.float32),
            pltpu.VMEM((_C,), jnp.int32),
            pltpu.VMEM((_C,), jnp.int32),
            pltpu.VMEM((_C, _HP), jnp.float32),
            pltpu.VMEM((_C, _HP), jnp.float32),
            pltpu.SemaphoreType.DMA,
        ],
        compiler_params=pltpu.CompilerParams(needs_layout_passes=False),
    )
    def k(h_hbm, w_hbm, src_hbm, dst_hbm, out_hbm, acc, si, di, rows, wv, sem):
        c = lax.axis_index("c")
        s = lax.axis_index("s")
        wid = s * _NC + c

        def zbody(i, carry):
            for j in range(_HP // 16):
                rows[i, pl.ds(j * 16, 16)] = jnp.zeros((16,), jnp.float32)
            return carry

        lax.fori_loop(0, _C, zbody, 0)
        for t in range(_RPT // _C):
            pltpu.sync_copy(rows, acc.at[pl.ds(s * _RPT + t * _C, _C), :])
        plsc.subcore_barrier()

        base = wid * _EW

        def chunk(kk, carry):
            off = base + kk * _C
            pltpu.sync_copy(src_hbm.at[pl.ds(off, _C)], si)
            pltpu.sync_copy(dst_hbm.at[pl.ds(off, _C)], di)
            pltpu.sync_copy(w_hbm.at[pl.ds(off, _C), :], wv)
            pltpu.async_copy(h_hbm.at[si], rows, sem).wait()

            def mbody(i, carry2):
                for j in range(_HP // 16):
                    sl = pl.ds(j * 16, 16)
                    rows[i, sl] = rows[i, sl] * wv[i, sl]
                return carry2

            lax.fori_loop(0, _C, mbody, 0)
            pltpu.sync_copy(rows, acc.at[di], add=True)
            return carry

        lax.fori_loop(0, _NCH, chunk, 0)
        plsc.subcore_barrier()
        pltpu.sync_copy(
            acc.at[pl.ds(s * _RPT, _RPT), :],
            out_hbm.at[c, pl.ds(s * _RPT, _RPT), :],
        )

    return k(h, wenv, src, dst)


def _h0_tc(x, W):
    def body(x_ref, w_ref, o_ref):
        o_ref[...] = jnp.dot(x_ref[...], w_ref[...], preferred_element_type=jnp.float32)

    return pl.pallas_call(
        body,
        grid=(10,),
        in_specs=[
            pl.BlockSpec((1000, _DIN), lambda i: (i, 0)),
            pl.BlockSpec((_DIN, _HP), lambda i: (0, 0)),
        ],
        out_specs=pl.BlockSpec((1000, _HP), lambda i: (i, 0)),
        out_shape=jax.ShapeDtypeStruct((_N, _HP), jnp.float32),
    )(x, W)


def _wenv_tc(d2, W1, b1, W2, b2, r_max):
    centers = [float(c) for c in np.linspace(0.0, r_max, _RBF)]
    width = r_max / _RBF
    inv_w2 = 1.0 / (width * width)
    B = 1280

    def body(d_ref, w1_ref, b1_ref, w2_ref, b2_ref, o_ref):
        d = jnp.sqrt(d_ref[...] + 1e-9)
        rbf = jnp.concatenate(
            [jnp.exp(-((d - c) ** 2) * inv_w2) for c in centers], axis=1
        )
        u = jax.nn.silu(
            jnp.dot(rbf, w1_ref[...], preferred_element_type=jnp.float32) + b1_ref[...]
        )
        w = jnp.dot(u, w2_ref[...], preferred_element_type=jnp.float32) + b2_ref[...]
        env = 0.5 * (jnp.cos(jnp.pi * d / r_max) + 1.0) * (d < r_max).astype(jnp.float32)
        o_ref[...] = w * env

    return pl.pallas_call(
        body,
        grid=(_E // B,),
        in_specs=[
            pl.BlockSpec((B, 1), lambda i: (i, 0)),
            pl.BlockSpec((_RBF, 64), lambda i: (0, 0)),
            pl.BlockSpec((1, 64), lambda i: (0, 0)),
            pl.BlockSpec((64, _HP), lambda i: (0, 0)),
            pl.BlockSpec((1, _HP), lambda i: (0, 0)),
        ],
        out_specs=pl.BlockSpec((B, _HP), lambda i: (i, 0)),
        out_shape=jax.ShapeDtypeStruct((_E, _HP), jnp.float32),
    )(d2, W1, b1, W2, b2)


def _node_tc(p0, p1, h, Wself, Wg, bg, Rm):
    B = 1000

    def body(p0_ref, p1_ref, h_ref, ws_ref, wg_ref, bg_ref, r_ref, o_ref):
        agg = p0_ref[...] + p1_ref[...]
        out = jnp.dot(agg, ws_ref[...], preferred_element_type=jnp.float32)
        g = jax.nn.sigmoid(
            jnp.dot(agg, wg_ref[...], preferred_element_type=jnp.float32) + bg_ref[...]
        )
        gf = jnp.dot(g, r_ref[...], preferred_element_type=jnp.float32)
        lanes = lax.broadcasted_iota(jnp.int32, out.shape, 1)
        o_ref[...] = h_ref[...] + jnp.where(lanes < _SCAL, jax.nn.silu(out), out * gf)

    full = lambda shape: pl.BlockSpec(shape, lambda i: (0, 0))
    return pl.pallas_call(
        body,
        grid=(_N // B,),
        in_specs=[
            pl.BlockSpec((B, _HP), lambda i: (i, 0)),
            pl.BlockSpec((B, _HP), lambda i: (i, 0)),
            pl.BlockSpec((B, _HP), lambda i: (i, 0)),
            full((_HP, _HP)),
            full((_HP, _HP)),
            full((1, _HP)),
            full((_HP, _HP)),
        ],
        out_specs=pl.BlockSpec((B, _HP), lambda i: (i, 0)),
        out_shape=jax.ShapeDtypeStruct((_N, _HP), jnp.float32),
    )(p0, p1, h, Wself, Wg, bg, Rm)


def _layernorm(z, g, b):
    mu = jnp.mean(z, axis=-1, keepdims=True)
    var = jnp.mean((z - mu) ** 2, axis=-1, keepdims=True)
    return (z - mu) / jnp.sqrt(var + 1e-5) * g + b


def _pool_tc(hs, batch2d, pW1, pb1, pW2, pb2, fW1, fb1, fg1, fbe1, fW2, fb2, fg2, fbe2):
    def body(hs_ref, b_ref, pw1_ref, pb1_ref, pw2_ref, pb2_ref, fw1_ref, fb1_ref,
             fg1_ref, fbe1_ref, fw2_ref, fb2_ref, fg2_ref, fbe2_ref, o_ref):
        bcol = b_ref[...]
        segs = lax.broadcasted_iota(jnp.int32, (_N, _NSEG), 1)
        M = (bcol == segs).astype(jnp.float32)
        pooled = []
        for s in range(3):
            hsc = hs_ref[s]
            u = jax.nn.silu(
                jnp.dot(hsc, pw1_ref[s], preferred_element_type=jnp.float32) + pb1_ref[s]
            )
            logits = jnp.dot(u, pw2_ref[s], preferred_element_type=jnp.float32) + pb2_ref[s]
            lg = jnp.where(M > 0, logits, -1e30)
            m = jnp.max(lg, axis=0, keepdims=True)
            mb = jnp.sum(M * m, axis=1, keepdims=True)
            ex = jnp.exp(logits - mb)
            den = jnp.sum(M * ex, axis=0, keepdims=True)
            denb = jnp.sum(M * den, axis=1, keepdims=True)
            attn = ex / (denb + 1e-16)
            pooled.append(
                lax.dot_general(M, hsc * attn, (((0,), (0,)), ((), ())),
                                preferred_element_type=jnp.float32)
            )
        comb = jnp.concatenate(pooled, axis=1)
        z = jnp.dot(comb, fw1_ref[...], preferred_element_type=jnp.float32) + fb1_ref[...]
        z = _layernorm(z, fg1_ref[...], fbe1_ref[...])
        z = jax.nn.silu(z)
        z = jnp.dot(z, fw2_ref[...], preferred_element_type=jnp.float32) + fb2_ref[...]
        o_ref[...] = _layernorm(z, fg2_ref[...], fbe2_ref[...])

    return pl.pallas_call(
        body,
        out_shape=jax.ShapeDtypeStruct((_NSEG, _OUT), jnp.float32),
    )(hs, batch2d, pW1, pb1, pW2, pb2, fW1, fb1, fg1, fbe1, fW2, fb2, fg2, fbe2)


def kernel(x, pos, params, edge_index, batch):
    p = params
    f32 = jnp.float32
    src = edge_index[0]
    dst = edge_index[1]

    Win_p = jnp.zeros((_DIN, _HP), f32).at[:, :_SCAL].set(p["W_in"])
    W2_p = jnp.zeros((6, 64, _HP), f32).at[:, :, :_HID].set(p["l_W2"])
    b2_p = jnp.zeros((6, 1, _HP), f32).at[:, 0, :_HID].set(p["l_b2"])
    Wself_p = jnp.zeros((6, _HP, _HP), f32).at[:, :_HID, :_HID].set(p["l_Wself"])
    Wg_p = jnp.zeros((6, _HP, _HP), f32).at[:, :_SCAL, :24].set(p["l_Wg"])
    bg_p = jnp.zeros((6, 1, _HP), f32).at[:, 0, :24].set(p["l_bg"])
    Rm = _gate_matrix()

    d2 = _dist2_sc(pos[:, 0], pos[:, 1], pos[:, 2], src, dst).reshape(_E, 1)
    h = _h0_tc(x, Win_p)

    hs_list = []
    li = 0
    for s, r_max in enumerate(_SCALES):
        hsc = h
        for _ in range(2):
            wenv = _wenv_tc(d2, p["l_W1"][li], p["l_b1"][li].reshape(1, 64),
                            W2_p[li], b2_p[li], r_max)
            parts = _edge_sc(hsc, wenv, src, dst)[:, :_N]
            hsc = _node_tc(parts[0], parts[1], hsc, Wself_p[li], Wg_p[li],
                           bg_p[li], Rm)
            li += 1
        hs_list.append(hsc[:, :_SCAL])
    hs = jnp.stack(hs_list)

    return _pool_tc(
        hs, batch.reshape(_N, 1),
        p["p_W1"], p["p_b1"].reshape(3, 1, 64), p["p_W2"], p["p_b2"].reshape(3, 1, 1),
        p["f_W1"], p["f_b1"].reshape(1, -1), p["f_g1"].reshape(1, -1),
        p["f_be1"].reshape(1, -1),
        p["f_W2"], p["f_b2"].reshape(1, -1), p["f_g2"].reshape(1, -1),
        p["f_be2"].reshape(1, -1),
    )


# speedup vs baseline: 3.1252x; 1.8784x over previous
import functools

import numpy as np
import jax
import jax.numpy as jnp
from jax import lax
from jax.experimental import pallas as pl
from jax.experimental.pallas import tpu as pltpu
from jax.experimental.pallas import tpu_sc as plsc

_N = 10000
_E = 320000
_DIN = 128
_HID = 120
_HP = 128
_SCAL = 32
_RBF = 8
_NSEG = 16
_OUT = 512
_SCALES = (3.0, 6.0, 10.0)
_REPS = (3,) * 16 + (5,) * 8

_NC = 2
_NS = 16
_NW = _NC * _NS
_EW = _E // _NW
_C = 80
_NCH = _EW // _C
_NP = 10240
_RPT = _NP // _NS


def _gate_matrix():
    R = np.zeros((_HP, _HP), np.float32)
    col = _SCAL
    for k, r in enumerate(_REPS):
        for _ in range(r):
            R[k, col] = 1.0
            col += 1
    return jnp.asarray(R)


def _dist2_sc(posx, posy, posz, src, dst):
    mesh = plsc.VectorSubcoreMesh(core_axis_name="c", subcore_axis_name="s")

    @functools.partial(
        pl.kernel,
        out_type=jax.ShapeDtypeStruct((_E,), jnp.float32),
        mesh=mesh,
        scratch_types=[
            pltpu.VMEM((_N,), jnp.float32),
            pltpu.VMEM((_N,), jnp.float32),
            pltpu.VMEM((_N,), jnp.float32),
            pltpu.VMEM((_EW,), jnp.int32),
            pltpu.VMEM((_EW,), jnp.int32),
            pltpu.VMEM((_EW,), jnp.float32),
        ],
        compiler_params=pltpu.CompilerParams(needs_layout_passes=False),
    )
    def k(px_hbm, py_hbm, pz_hbm, src_hbm, dst_hbm, out_hbm, px, py, pz, sv, dv, ov):
        c = lax.axis_index("c")
        s = lax.axis_index("s")
        wid = s * _NC + c
        base = wid * _EW
        pltpu.sync_copy(px_hbm, px)
        pltpu.sync_copy(py_hbm, py)
        pltpu.sync_copy(pz_hbm, pz)
        pltpu.sync_copy(src_hbm.at[pl.ds(base, _EW)], sv)
        pltpu.sync_copy(dst_hbm.at[pl.ds(base, _EW)], dv)

        def body(i, carry):
            off = i * 16
            s16 = sv[pl.ds(off, 16)]
            d16 = dv[pl.ds(off, 16)]
            dx = plsc.load_gather(px, [d16]) - plsc.load_gather(px, [s16])
            dy = plsc.load_gather(py, [d16]) - plsc.load_gather(py, [s16])
            dz = plsc.load_gather(pz, [d16]) - plsc.load_gather(pz, [s16])
            ov[pl.ds(off, 16)] = dx * dx + dy * dy + dz * dz
            return carry

        lax.fori_loop(0, _EW // 16, body, 0)
        pltpu.sync_copy(ov, out_hbm.at[pl.ds(base, _EW)])

    return k(posx, posy, posz, src, dst)


def _edge_sc(h, wenv, src, dst):
    mesh = plsc.VectorSubcoreMesh(core_axis_name="c", subcore_axis_name="s")

    @functools.partial(
        pl.kernel,
        out_type=jax.ShapeDtypeStruct((_NC, _NP, _HP), jnp.float32),
        mesh=mesh,
        scratch_types=[
            pltpu.VMEM_SHARED((_NP, _HP), jnp.float32),
            pltpu.VMEM((_C,), jnp.int32),
            pltpu.VMEM((_C,), jnp.int32),
            pltpu.VMEM((_C, _HP), jnp.float32),
            pltpu.VMEM((_C, _HP), jnp.float32),
            pltpu.SemaphoreType.DMA,
        ],
        compiler_params=pltpu.CompilerParams(needs_layout_passes=False),
    )
    def k(h_hbm, w_hbm, src_hbm, dst_hbm, out_hbm, acc, si, di, rows, wv, sem):
        c = lax.axis_index("c")
        s = lax.axis_index("s")
        wid = s * _NC + c

        def zbody(i, carry):
            for j in range(_HP // 16):
                rows[i, pl.ds(j * 16, 16)] = jnp.zeros((16,), jnp.float32)
            return carry

        lax.fori_loop(0, _C, zbody, 0)
        for t in range(_RPT // _C):
            pltpu.sync_copy(rows, acc.at[pl.ds(s * _RPT + t * _C, _C), :])
        plsc.subcore_barrier()

        base = wid * _EW

        def chunk(kk, carry):
            off = base + kk * _C
            pltpu.sync_copy(src_hbm.at[pl.ds(off, _C)], si)
            pltpu.sync_copy(dst_hbm.at[pl.ds(off, _C)], di)
            pltpu.sync_copy(w_hbm.at[pl.ds(off, _C), :], wv)
            pltpu.async_copy(h_hbm.at[si], rows, sem).wait()

            def mbody(i, carry2):
                for j in range(_HP // 16):
                    sl = pl.ds(j * 16, 16)
                    rows[i, sl] = rows[i, sl] * wv[i, sl]
                return carry2

            lax.fori_loop(0, _C, mbody, 0)
            pltpu.sync_copy(rows, acc.at[di], add=True)
            return carry

        lax.fori_loop(0, _NCH, chunk, 0)
        plsc.subcore_barrier()
        pltpu.sync_copy(
            acc.at[pl.ds(s * _RPT, _RPT), :],
            out_hbm.at[c, pl.ds(s * _RPT, _RPT), :],
        )

    return k(h, wenv, src, dst)


def _h0_tc(x, W):
    def body(x_ref, w_ref, o_ref):
        o_ref[...] = jnp.dot(x_ref[...], w_ref[...], preferred_element_type=jnp.float32)

    return pl.pallas_call(
        body,
        grid=(10,),
        in_specs=[
            pl.BlockSpec((1000, _DIN), lambda i: (i, 0)),
            pl.BlockSpec((_DIN, _HP), lambda i: (0, 0)),
        ],
        out_specs=pl.BlockSpec((1000, _HP), lambda i: (i, 0)),
        out_shape=jax.ShapeDtypeStruct((_N, _HP), jnp.float32),
    )(x, W)


def _wenv_tc(d2row, W1, b1, W2, b2, r_max):
    width = r_max / _RBF
    inv_w2 = 1.0 / (width * width)
    step = r_max / (_RBF - 1)
    B = 1280

    def body(d_ref, w1_ref, b1_ref, w2_ref, b2_ref, o_ref):
        dr = jnp.sqrt(d_ref[...] + 1e-9)
        cen = lax.broadcasted_iota(jnp.int32, (_RBF, 1), 0).astype(jnp.float32) * step
        rbf_t = jnp.exp(-((dr - cen) ** 2) * inv_w2)
        env_t = 0.5 * (jnp.cos(jnp.pi / r_max * dr) + 1.0) * (dr < r_max).astype(
            jnp.float32
        )
        t = jnp.transpose(jnp.concatenate([rbf_t, env_t], axis=0))
        rbf = t[:, :_RBF]
        env = t[:, _RBF:_RBF + 1]
        u = jax.nn.silu(
            jnp.dot(rbf, w1_ref[...], preferred_element_type=jnp.float32) + b1_ref[...]
        )
        w = jnp.dot(u, w2_ref[...], preferred_element_type=jnp.float32) + b2_ref[...]
        o_ref[...] = w * env

    return pl.pallas_call(
        body,
        grid=(_E // B,),
        in_specs=[
            pl.BlockSpec((1, B), lambda i: (0, i)),
            pl.BlockSpec((_RBF, 64), lambda i: (0, 0)),
            pl.BlockSpec((1, 64), lambda i: (0, 0)),
            pl.BlockSpec((64, _HP), lambda i: (0, 0)),
            pl.BlockSpec((1, _HP), lambda i: (0, 0)),
        ],
        out_specs=pl.BlockSpec((B, _HP), lambda i: (i, 0)),
        out_shape=jax.ShapeDtypeStruct((_E, _HP), jnp.float32),
    )(d2row, W1, b1, W2, b2)


def _node_tc(p0, p1, h, Wself, Wg, bg, Rm):
    B = 1000

    def body(p0_ref, p1_ref, h_ref, ws_ref, wg_ref, bg_ref, r_ref, o_ref):
        agg = p0_ref[...] + p1_ref[...]
        out = jnp.dot(agg, ws_ref[...], preferred_element_type=jnp.float32)
        g = jax.nn.sigmoid(
            jnp.dot(agg, wg_ref[...], preferred_element_type=jnp.float32) + bg_ref[...]
        )
        gf = jnp.dot(g, r_ref[...], preferred_element_type=jnp.float32)
        lanes = lax.broadcasted_iota(jnp.int32, out.shape, 1)
        o_ref[...] = h_ref[...] + jnp.where(lanes < _SCAL, jax.nn.silu(out), out * gf)

    full = lambda shape: pl.BlockSpec(shape, lambda i: (0, 0))
    return pl.pallas_call(
        body,
        grid=(_N // B,),
        in_specs=[
            pl.BlockSpec((B, _HP), lambda i: (i, 0)),
            pl.BlockSpec((B, _HP), lambda i: (i, 0)),
            pl.BlockSpec((B, _HP), lambda i: (i, 0)),
            full((_HP, _HP)),
            full((_HP, _HP)),
            full((1, _HP)),
            full((_HP, _HP)),
        ],
        out_specs=pl.BlockSpec((B, _HP), lambda i: (i, 0)),
        out_shape=jax.ShapeDtypeStruct((_N, _HP), jnp.float32),
    )(p0, p1, h, Wself, Wg, bg, Rm)


def _layernorm(z, g, b):
    mu = jnp.mean(z, axis=-1, keepdims=True)
    var = jnp.mean((z - mu) ** 2, axis=-1, keepdims=True)
    return (z - mu) / jnp.sqrt(var + 1e-5) * g + b


def _pool_tc(hs, batch2d, pW1, pb1, pW2, pb2, fW1, fb1, fg1, fbe1, fW2, fb2, fg2, fbe2):
    def body(hs_ref, b_ref, pw1_ref, pb1_ref, pw2_ref, pb2_ref, fw1_ref, fb1_ref,
             fg1_ref, fbe1_ref, fw2_ref, fb2_ref, fg2_ref, fbe2_ref, o_ref):
        bcol = b_ref[...]
        segs = lax.broadcasted_iota(jnp.int32, (_N, _NSEG), 1)
        M = (bcol == segs).astype(jnp.float32)
        pooled = []
        for s in range(3):
            hsc = hs_ref[s]
            u = jax.nn.silu(
                jnp.dot(hsc, pw1_ref[s], preferred_element_type=jnp.float32) + pb1_ref[s]
            )
            logits = jnp.dot(u, pw2_ref[s], preferred_element_type=jnp.float32) + pb2_ref[s]
            lg = jnp.where(M > 0, logits, -1e30)
            m = jnp.max(lg, axis=0, keepdims=True)
            mb = jnp.sum(M * m, axis=1, keepdims=True)
            ex = jnp.exp(logits - mb)
            den = jnp.sum(M * ex, axis=0, keepdims=True)
            denb = jnp.sum(M * den, axis=1, keepdims=True)
            attn = ex / (denb + 1e-16)
            pooled.append(
                lax.dot_general(M, hsc * attn, (((0,), (0,)), ((), ())),
                                preferred_element_type=jnp.float32)
            )
        comb = jnp.concatenate(pooled, axis=1)
        z = jnp.dot(comb, fw1_ref[...], preferred_element_type=jnp.float32) + fb1_ref[...]
        z = _layernorm(z, fg1_ref[...], fbe1_ref[...])
        z = jax.nn.silu(z)
        z = jnp.dot(z, fw2_ref[...], preferred_element_type=jnp.float32) + fb2_ref[...]
        o_ref[...] = _layernorm(z, fg2_ref[...], fbe2_ref[...])

    return pl.pallas_call(
        body,
        out_shape=jax.ShapeDtypeStruct((_NSEG, _OUT), jnp.float32),
    )(hs, batch2d, pW1, pb1, pW2, pb2, fW1, fb1, fg1, fbe1, fW2, fb2, fg2, fbe2)


def kernel(x, pos, params, edge_index, batch):
    p = params
    f32 = jnp.float32
    src = edge_index[0]
    dst = edge_index[1]

    Win_p = jnp.zeros((_DIN, _HP), f32).at[:, :_SCAL].set(p["W_in"])
    W2_p = jnp.zeros((6, 64, _HP), f32).at[:, :, :_HID].set(p["l_W2"])
    b2_p = jnp.zeros((6, 1, _HP), f32).at[:, 0, :_HID].set(p["l_b2"])
    Wself_p = jnp.zeros((6, _HP, _HP), f32).at[:, :_HID, :_HID].set(p["l_Wself"])
    Wg_p = jnp.zeros((6, _HP, _HP), f32).at[:, :_SCAL, :24].set(p["l_Wg"])
    bg_p = jnp.zeros((6, 1, _HP), f32).at[:, 0, :24].set(p["l_bg"])
    Rm = _gate_matrix()

    d2 = _dist2_sc(pos[:, 0], pos[:, 1], pos[:, 2], src, dst).reshape(1, _E)
    h = _h0_tc(x, Win_p)

    hs_list = []
    li = 0
    for s, r_max in enumerate(_SCALES):
        hsc = h
        for _ in range(2):
            wenv = _wenv_tc(d2, p["l_W1"][li], p["l_b1"][li].reshape(1, 64),
                            W2_p[li], b2_p[li], r_max)
            parts = _edge_sc(hsc, wenv, src, dst)[:, :_N]
            hsc = _node_tc(parts[0], parts[1], hsc, Wself_p[li], Wg_p[li],
                           bg_p[li], Rm)
            li += 1
        hs_list.append(hsc[:, :_SCAL])
    hs = jnp.stack(hs_list)

    return _pool_tc(
        hs, batch.reshape(_N, 1),
        p["p_W1"], p["p_b1"].reshape(3, 1, 64), p["p_W2"], p["p_b2"].reshape(3, 1, 1),
        p["f_W1"], p["f_b1"].reshape(1, -1), p["f_g1"].reshape(1, -1),
        p["f_be1"].reshape(1, -1),
        p["f_W2"], p["f_b2"].reshape(1, -1), p["f_g2"].reshape(1, -1),
        p["f_be2"].reshape(1, -1),
    )


# speedup vs baseline: 5.9472x; 1.9030x over previous
import functools

import numpy as np
import jax
import jax.numpy as jnp
from jax import lax
from jax.experimental import pallas as pl
from jax.experimental.pallas import tpu as pltpu
from jax.experimental.pallas import tpu_sc as plsc

_N = 10000
_E = 320000
_DIN = 128
_HID = 120
_HP = 128
_SCAL = 32
_RBF = 8
_NSEG = 16
_OUT = 512
_SCALES = (3.0, 6.0, 10.0)
_REPS = (3,) * 16 + (5,) * 8

_NC = 2
_NS = 16
_NW = _NC * _NS
_EW = _E // _NW
_C = 80
_NCH = _EW // _C
_NP = 10240
_RPT = _NP // _NS


def _gate_matrix():
    R = np.zeros((_HP, _HP), np.float32)
    col = _SCAL
    for k, r in enumerate(_REPS):
        for _ in range(r):
            R[k, col] = 1.0
            col += 1
    return jnp.asarray(R)


def _dist2_sc(posx, posy, posz, src, dst):
    mesh = plsc.VectorSubcoreMesh(core_axis_name="c", subcore_axis_name="s")

    @functools.partial(
        pl.kernel,
        out_type=jax.ShapeDtypeStruct((_E,), jnp.float32),
        mesh=mesh,
        scratch_types=[
            pltpu.VMEM((_N,), jnp.float32),
            pltpu.VMEM((_N,), jnp.float32),
            pltpu.VMEM((_N,), jnp.float32),
            pltpu.VMEM((_EW,), jnp.int32),
            pltpu.VMEM((_EW,), jnp.int32),
            pltpu.VMEM((_EW,), jnp.float32),
        ],
        compiler_params=pltpu.CompilerParams(needs_layout_passes=False),
    )
    def k(px_hbm, py_hbm, pz_hbm, src_hbm, dst_hbm, out_hbm, px, py, pz, sv, dv, ov):
        c = lax.axis_index("c")
        s = lax.axis_index("s")
        wid = s * _NC + c
        base = wid * _EW
        pltpu.sync_copy(px_hbm, px)
        pltpu.sync_copy(py_hbm, py)
        pltpu.sync_copy(pz_hbm, pz)
        pltpu.sync_copy(src_hbm.at[pl.ds(base, _EW)], sv)
        pltpu.sync_copy(dst_hbm.at[pl.ds(base, _EW)], dv)

        def body(i, carry):
            off = i * 16
            s16 = sv[pl.ds(off, 16)]
            d16 = dv[pl.ds(off, 16)]
            dx = plsc.load_gather(px, [d16]) - plsc.load_gather(px, [s16])
            dy = plsc.load_gather(py, [d16]) - plsc.load_gather(py, [s16])
            dz = plsc.load_gather(pz, [d16]) - plsc.load_gather(pz, [s16])
            ov[pl.ds(off, 16)] = dx * dx + dy * dy + dz * dz
            return carry

        lax.fori_loop(0, _EW // 16, body, 0)
        pltpu.sync_copy(ov, out_hbm.at[pl.ds(base, _EW)])

    return k(posx, posy, posz, src, dst)


def _edge_sc(h, wenv, src, dst):
    mesh = plsc.VectorSubcoreMesh(core_axis_name="c", subcore_axis_name="s")

    @functools.partial(
        pl.kernel,
        out_type=jax.ShapeDtypeStruct((_NC, _NP, _HP), jnp.float32),
        mesh=mesh,
        scratch_types=[
            pltpu.VMEM_SHARED((_NP, _HP), jnp.float32),
            pltpu.VMEM((_C,), jnp.int32),
            pltpu.VMEM((_C,), jnp.int32),
            pltpu.VMEM((_C, _HP), jnp.float32),
            pltpu.VMEM((_C, _HP), jnp.float32),
            pltpu.VMEM((_C,), jnp.int32),
            pltpu.VMEM((_C,), jnp.int32),
            pltpu.VMEM((_C, _HP), jnp.float32),
            pltpu.VMEM((_C, _HP), jnp.float32),
            pltpu.SemaphoreType.DMA,
            pltpu.SemaphoreType.DMA,
            pltpu.SemaphoreType.DMA,
            pltpu.SemaphoreType.DMA,
        ],
        compiler_params=pltpu.CompilerParams(needs_layout_passes=False),
    )
    def k(h_hbm, w_hbm, src_hbm, dst_hbm, out_hbm,
          acc, si0, di0, rows0, wv0, si1, di1, rows1, wv1,
          semi0, semi1, semg0, semg1):
        c = lax.axis_index("c")
        s = lax.axis_index("s")
        wid = s * _NC + c
        base = wid * _EW
        bufs = ((si0, di0, rows0, wv0, semi0, semg0),
                (si1, di1, rows1, wv1, semi1, semg1))

        def issue_in(kk, b):
            si_, di_, _, wv_, semi, _ = bufs[b]
            off = base + kk * _C
            pltpu.async_copy(src_hbm.at[pl.ds(off, _C)], si_, semi)
            pltpu.async_copy(dst_hbm.at[pl.ds(off, _C)], di_, semi)
            pltpu.async_copy(w_hbm.at[pl.ds(off, _C), :], wv_, semi)

        def wait_in(b):
            si_, di_, _, wv_, semi, _ = bufs[b]
            pltpu.make_async_copy(src_hbm.at[pl.ds(0, _C)], si_, semi).wait()
            pltpu.make_async_copy(dst_hbm.at[pl.ds(0, _C)], di_, semi).wait()
            pltpu.make_async_copy(w_hbm.at[pl.ds(0, _C), :], wv_, semi).wait()

        def issue_g(b):
            si_, _, rows_, _, _, semg = bufs[b]
            return pltpu.async_copy(h_hbm.at[si_], rows_, semg)

        def mult_scatter(b):
            _, di_, rows_, wv_, _, _ = bufs[b]

            def mbody(i, carry2):
                for j in range(_HP // 16):
                    sl = pl.ds(j * 16, 16)
                    rows_[i, sl] = rows_[i, sl] * wv_[i, sl]
                return carry2

            lax.fori_loop(0, _C, mbody, 0)
            pltpu.sync_copy(rows_, acc.at[di_], add=True)

        def zbody(i, carry):
            for j in range(_HP // 16):
                rows0[i, pl.ds(j * 16, 16)] = jnp.zeros((16,), jnp.float32)
            return carry

        lax.fori_loop(0, _C, zbody, 0)
        for t in range(_RPT // _C):
            pltpu.sync_copy(rows0, acc.at[pl.ds(s * _RPT + t * _C, _C), :])
        plsc.subcore_barrier()

        issue_in(0, 0)
        issue_in(1, 1)

        def step(t, carry):
            k0 = 2 * t
            wait_in(0)
            g0 = issue_g(0)
            wait_in(1)
            g1 = issue_g(1)
            g0.wait()
            mult_scatter(0)
            issue_in(k0 + 2, 0)
            g1.wait()
            mult_scatter(1)

            @pl.when(k0 + 3 < _NCH)
            def _():
                issue_in(k0 + 3, 1)

            return carry

        lax.fori_loop(0, (_NCH - 1) // 2, step, 0)
        wait_in(0)
        issue_g(0).wait()
        mult_scatter(0)
        plsc.subcore_barrier()
        pltpu.sync_copy(
            acc.at[pl.ds(s * _RPT, _RPT), :],
            out_hbm.at[c, pl.ds(s * _RPT, _RPT), :],
        )

    return k(h, wenv, src, dst)


def _h0_tc(x, W):
    def body(x_ref, w_ref, o_ref):
        o_ref[...] = jnp.dot(x_ref[...], w_ref[...], preferred_element_type=jnp.float32)

    return pl.pallas_call(
        body,
        grid=(10,),
        in_specs=[
            pl.BlockSpec((1000, _DIN), lambda i: (i, 0)),
            pl.BlockSpec((_DIN, _HP), lambda i: (0, 0)),
        ],
        out_specs=pl.BlockSpec((1000, _HP), lambda i: (i, 0)),
        out_shape=jax.ShapeDtypeStruct((_N, _HP), jnp.float32),
    )(x, W)


def _wenv_tc(d2row, W1, b1, W2, b2, r_max):
    width = r_max / _RBF
    inv_w2 = 1.0 / (width * width)
    step = r_max / (_RBF - 1)
    B = 1280

    def body(d_ref, w1_ref, b1_ref, w2_ref, b2_ref, o_ref):
        dr = jnp.sqrt(d_ref[...] + 1e-9)
        cen = lax.broadcasted_iota(jnp.int32, (_RBF, 1), 0).astype(jnp.float32) * step
        rbf_t = jnp.exp(-((dr - cen) ** 2) * inv_w2)
        env_t = 0.5 * (jnp.cos(jnp.pi / r_max * dr) + 1.0) * (dr < r_max).astype(
            jnp.float32
        )
        t = jnp.transpose(jnp.concatenate([rbf_t, env_t], axis=0))
        rbf = t[:, :_RBF]
        env = t[:, _RBF:_RBF + 1]
        u = jax.nn.silu(
            jnp.dot(rbf, w1_ref[...], preferred_element_type=jnp.float32) + b1_ref[...]
        )
        w = jnp.dot(u, w2_ref[...], preferred_element_type=jnp.float32) + b2_ref[...]
        o_ref[...] = w * env

    return pl.pallas_call(
        body,
        grid=(_E // B,),
        in_specs=[
            pl.BlockSpec((1, B), lambda i: (0, i)),
            pl.BlockSpec((_RBF, 64), lambda i: (0, 0)),
            pl.BlockSpec((1, 64), lambda i: (0, 0)),
            pl.BlockSpec((64, _HP), lambda i: (0, 0)),
            pl.BlockSpec((1, _HP), lambda i: (0, 0)),
        ],
        out_specs=pl.BlockSpec((B, _HP), lambda i: (i, 0)),
        out_shape=jax.ShapeDtypeStruct((_E, _HP), jnp.float32),
    )(d2row, W1, b1, W2, b2)


def _node_tc(p0, p1, h, Wself, Wg, bg, Rm):
    B = 1000

    def body(p0_ref, p1_ref, h_ref, ws_ref, wg_ref, bg_ref, r_ref, o_ref):
        agg = p0_ref[...] + p1_ref[...]
        out = jnp.dot(agg, ws_ref[...], preferred_element_type=jnp.float32)
        g = jax.nn.sigmoid(
            jnp.dot(agg, wg_ref[...], preferred_element_type=jnp.float32) + bg_ref[...]
        )
        gf = jnp.dot(g, r_ref[...], preferred_element_type=jnp.float32)
        lanes = lax.broadcasted_iota(jnp.int32, out.shape, 1)
        o_ref[...] = h_ref[...] + jnp.where(lanes < _SCAL, jax.nn.silu(out), out * gf)

    full = lambda shape: pl.BlockSpec(shape, lambda i: (0, 0))
    return pl.pallas_call(
        body,
        grid=(_N // B,),
        in_specs=[
            pl.BlockSpec((B, _HP), lambda i: (i, 0)),
            pl.BlockSpec((B, _HP), lambda i: (i, 0)),
            pl.BlockSpec((B, _HP), lambda i: (i, 0)),
            full((_HP, _HP)),
            full((_HP, _HP)),
            full((1, _HP)),
            full((_HP, _HP)),
        ],
        out_specs=pl.BlockSpec((B, _HP), lambda i: (i, 0)),
        out_shape=jax.ShapeDtypeStruct((_N, _HP), jnp.float32),
    )(p0, p1, h, Wself, Wg, bg, Rm)


def _layernorm(z, g, b):
    mu = jnp.mean(z, axis=-1, keepdims=True)
    var = jnp.mean((z - mu) ** 2, axis=-1, keepdims=True)
    return (z - mu) / jnp.sqrt(var + 1e-5) * g + b


def _pool_tc(hs, batch2d, pW1, pb1, pW2, pb2, fW1, fb1, fg1, fbe1, fW2, fb2, fg2, fbe2):
    def body(hs_ref, b_ref, pw1_ref, pb1_ref, pw2_ref, pb2_ref, fw1_ref, fb1_ref,
             fg1_ref, fbe1_ref, fw2_ref, fb2_ref, fg2_ref, fbe2_ref, o_ref):
        bcol = b_ref[...]
        segs = lax.broadcasted_iota(jnp.int32, (_N, _NSEG), 1)
        M = (bcol == segs).astype(jnp.float32)
        pooled = []
        for s in range(3):
            hsc = hs_ref[s]
            u = jax.nn.silu(
                jnp.dot(hsc, pw1_ref[s], preferred_element_type=jnp.float32) + pb1_ref[s]
            )
            logits = jnp.dot(u, pw2_ref[s], preferred_element_type=jnp.float32) + pb2_ref[s]
            lg = jnp.where(M > 0, logits, -1e30)
            m = jnp.max(lg, axis=0, keepdims=True)
            mb = jnp.sum(M * m, axis=1, keepdims=True)
            ex = jnp.exp(logits - mb)
            den = jnp.sum(M * ex, axis=0, keepdims=True)
            denb = jnp.sum(M * den, axis=1, keepdims=True)
            attn = ex / (denb + 1e-16)
            pooled.append(
                lax.dot_general(M, hsc * attn, (((0,), (0,)), ((), ())),
                                preferred_element_type=jnp.float32)
            )
        comb = jnp.concatenate(pooled, axis=1)
        z = jnp.dot(comb, fw1_ref[...], preferred_element_type=jnp.float32) + fb1_ref[...]
        z = _layernorm(z, fg1_ref[...], fbe1_ref[...])
        z = jax.nn.silu(z)
        z = jnp.dot(z, fw2_ref[...], preferred_element_type=jnp.float32) + fb2_ref[...]
        o_ref[...] = _layernorm(z, fg2_ref[...], fbe2_ref[...])

    return pl.pallas_call(
        body,
        out_shape=jax.ShapeDtypeStruct((_NSEG, _OUT), jnp.float32),
    )(hs, batch2d, pW1, pb1, pW2, pb2, fW1, fb1, fg1, fbe1, fW2, fb2, fg2, fbe2)


def kernel(x, pos, params, edge_index, batch):
    p = params
    f32 = jnp.float32
    src = edge_index[0]
    dst = edge_index[1]

    Win_p = jnp.zeros((_DIN, _HP), f32).at[:, :_SCAL].set(p["W_in"])
    W2_p = jnp.zeros((6, 64, _HP), f32).at[:, :, :_HID].set(p["l_W2"])
    b2_p = jnp.zeros((6, 1, _HP), f32).at[:, 0, :_HID].set(p["l_b2"])
    Wself_p = jnp.zeros((6, _HP, _HP), f32).at[:, :_HID, :_HID].set(p["l_Wself"])
    Wg_p = jnp.zeros((6, _HP, _HP), f32).at[:, :_SCAL, :24].set(p["l_Wg"])
    bg_p = jnp.zeros((6, 1, _HP), f32).at[:, 0, :24].set(p["l_bg"])
    Rm = _gate_matrix()

    d2 = _dist2_sc(pos[:, 0], pos[:, 1], pos[:, 2], src, dst).reshape(1, _E)
    h = _h0_tc(x, Win_p)

    hs_list = []
    li = 0
    for s, r_max in enumerate(_SCALES):
        hsc = h
        for _ in range(2):
            wenv = _wenv_tc(d2, p["l_W1"][li], p["l_b1"][li].reshape(1, 64),
                            W2_p[li], b2_p[li], r_max)
            parts = _edge_sc(hsc, wenv, src, dst)[:, :_N]
            hsc = _node_tc(parts[0], parts[1], hsc, Wself_p[li], Wg_p[li],
                           bg_p[li], Rm)
            li += 1
        hs_list.append(hsc[:, :_SCAL])
    hs = jnp.stack(hs_list)

    return _pool_tc(
        hs, batch.reshape(_N, 1),
        p["p_W1"], p["p_b1"].reshape(3, 1, 64), p["p_W2"], p["p_b2"].reshape(3, 1, 1),
        p["f_W1"], p["f_b1"].reshape(1, -1), p["f_g1"].reshape(1, -1),
        p["f_be1"].reshape(1, -1),
        p["f_W2"], p["f_b2"].reshape(1, -1), p["f_g2"].reshape(1, -1),
        p["f_be2"].reshape(1, -1),
    )
